# baseline (speedup 1.0000x reference)
import functools

import jax
import jax.numpy as jnp
import numpy as np
from jax.experimental import pallas as pl
from jax.experimental.pallas import tpu as pltpu

_NC = 20
_NA = 5
_ANCHORS = np.array(
    [1.3221, 1.73145, 3.19275, 4.00944, 5.05587, 8.09892, 9.47112, 4.84053,
     11.2364, 10.0071],
    dtype=np.float32).reshape(_NA, 2)
_THRESH = 0.6
_AREA_FRAC = _THRESH / (1.0 + _THRESH)
_NH = 19
_NW = 19
_NPIX = _NH * _NW
_NT = 50
_NCH = 5 + _NC


def _iou(b1x, b1y, b1w, b1h, b2x, b2y, b2w, b2h):
    mx = jnp.minimum(b1x - b1w / 2.0, b2x - b2w / 2.0)
    Mx = jnp.maximum(b1x + b1w / 2.0, b2x + b2w / 2.0)
    my = jnp.minimum(b1y - b1h / 2.0, b2y - b2h / 2.0)
    My = jnp.maximum(b1y + b1h / 2.0, b2y + b2h / 2.0)
    uw = Mx - mx
    uh = My - my
    cw = b1w + b2w - uw
    ch = b1h + b2h - uh
    carea = jnp.where((cw <= 0) | (ch <= 0), 0.0, cw * ch)
    uarea = b1w * b1h + b2w * b2h - carea
    return carea / uarea


def _const_vec(vals, shape, dim):
    idx = jax.lax.broadcasted_iota(jnp.int32, shape, dim)
    out = jnp.zeros(shape, jnp.float32)
    for i, v in enumerate(vals):
        out = jnp.where(idx == i, float(v), out)
    return out


def _region_loss_kernel(x_ref, tb_ref, tbt_ref, out_ref):
    x = x_ref[0]
    tb = tb_ref[0]
    tbt = tbt_ref[0]

    f32 = jnp.float32

    xs_c = tb[:, 1:2]
    ys_c = tb[:, 2:3]
    ws_c = tb[:, 3:4]
    hs_c = tb[:, 4:5]
    cls_c = tb[:, 0:1]
    gx_c = xs_c * _NW
    gy_c = ys_c * _NH
    gw_c = ws_c * _NW
    gh_c = hs_c * _NH
    gi_c = gx_c.astype(jnp.int32)
    gj_c = gy_c.astype(jnp.int32)

    xs_r = tbt[1:2, :]
    gx_r = xs_r * _NW
    gy_r = tbt[2:3, :] * _NH
    gw_r = tbt[3:4, :] * _NW
    gh_r = tbt[4:5, :] * _NH
    gi_r = gx_r.astype(jnp.int32)
    gj_r = gy_r.astype(jnp.int32)

    ti = jax.lax.broadcasted_iota(jnp.int32, (_NT, _NT), 0)
    tj = jax.lax.broadcasted_iota(jnp.int32, (_NT, _NT), 1)
    zeros_r = (xs_r == 0.0).astype(f32)
    zeros_c = (xs_c == 0.0).astype(f32)
    cnt_c = jnp.sum(jnp.where(tj <= ti, zeros_r, 0.0), axis=1, keepdims=True)
    valid_c = cnt_c == 0.0
    cnt_r = jnp.sum(jnp.where(ti <= tj, zeros_c, 0.0), axis=0, keepdims=True)
    valid_r = cnt_r == 0.0

    aw_r = _const_vec(_ANCHORS[:, 0], (1, _NA), 1)
    ah_r = _const_vec(_ANCHORS[:, 1], (1, _NA), 1)
    aw_c = _const_vec(_ANCHORS[:, 0], (_NA, 1), 0)
    ah_c = _const_vec(_ANCHORS[:, 1], (_NA, 1), 0)
    z = jnp.zeros((), f32)
    an_idx_r = jax.lax.broadcasted_iota(jnp.int32, (_NT, _NA), 1)
    tmp_c = _iou(z, z, aw_r, ah_r, z, z, gw_c, gh_c)
    m_c = jnp.max(tmp_c, axis=1, keepdims=True)
    bn_c = jnp.min(jnp.where(tmp_c == m_c, an_idx_r, _NA), axis=1,
                   keepdims=True)
    an_idx_c = jax.lax.broadcasted_iota(jnp.int32, (_NA, _NT), 0)
    tmp_r = _iou(z, z, aw_c, ah_c, z, z, gw_r, gh_r)
    m_r = jnp.max(tmp_r, axis=0, keepdims=True)
    bn_r = jnp.min(jnp.where(tmp_r == m_r, an_idx_c, _NA), axis=0,
                   keepdims=True)

    c_c = bn_c * _NPIX + gj_c * _NW + gi_c
    c_r = bn_r * _NPIX + gj_r * _NW + gi_r
    conflict = jnp.sum(
        jnp.where((tj > ti) & valid_r & (c_r == c_c), 1.0, 0.0),
        axis=1, keepdims=True)
    winner = valid_c & (conflict == 0.0)

    onehot_bn = (an_idx_r == bn_c).astype(f32)
    awn = jnp.sum(onehot_bn * aw_r, axis=1, keepdims=True)
    ahn = jnp.sum(onehot_bn * ah_r, axis=1, keepdims=True)

    tc0 = gx_c - gi_c.astype(f32)
    tc1 = gy_c - gj_c.astype(f32)
    tc2 = jnp.log(gw_c / awn)
    tc3 = jnp.log(gh_c / ahn)
    cm = 2.0 - ws_c * hs_c
    cls_idx = cls_c.astype(jnp.int32)
    cidx_r = jax.lax.broadcasted_iota(jnp.int32, (_NT, _NC), 1)
    onehot_cls = (cidx_r == cls_idx).astype(f32)

    gx1 = jnp.where(valid_c, gx_c - gw_c * 0.5, 3.0e38)
    gx2 = gx_c + gw_c * 0.5
    gy1 = gy_c - gh_c * 0.5
    gy2 = gy_c + gh_c * 0.5
    rhs_g = _AREA_FRAC * (gw_c * gh_c)

    col = jax.lax.broadcasted_iota(jnp.int32, (1, _NPIX), 1)
    grid_x = (col % _NW).astype(f32)
    grid_y = (col // _NW).astype(f32)
    p_r = jax.lax.broadcasted_iota(jnp.int32, (_NT, _NPIX), 1)

    conf_base = jnp.zeros((), f32)
    coord_base = jnp.zeros((), f32)
    raw_sel = jnp.zeros((_NT, _NCH), f32)
    ign_sel = jnp.zeros((_NT, 1), f32)

    for a in range(_NA):
        base = a * _NCH
        tx = x[base + 0:base + 1, :]
        ty = x[base + 1:base + 2, :]
        tw = x[base + 2:base + 3, :]
        th = x[base + 3:base + 4, :]
        cf = x[base + 4:base + 5, :]

        sx = jax.nn.sigmoid(tx)
        sy = jax.nn.sigmoid(ty)
        conf = jax.nn.sigmoid(cf)
        bx = sx + grid_x
        by = sy + grid_y
        hbw = jnp.exp(tw) * (0.5 * float(_ANCHORS[a, 0]))
        hbh = jnp.exp(th) * (0.5 * float(_ANCHORS[a, 1]))
        px1 = bx - hbw
        px2 = bx + hbw
        py1 = by - hbh
        py2 = by + hbh
        rhs_p = (_AREA_FRAC * 4.0) * (hbw * hbh)

        coord_base += (jnp.sum((sx - 0.5) ** 2) + jnp.sum((sy - 0.5) ** 2)
                       + jnp.sum(tw * tw) + jnp.sum(th * th))

        cw = jnp.minimum(px2, gx2) - jnp.maximum(px1, gx1)
        ch = jnp.minimum(py2, gy2) - jnp.maximum(py1, gy1)
        carea = jnp.maximum(cw, 0.0) * jnp.maximum(ch, 0.0)
        hit = carea > (rhs_p + rhs_g)
        ign = jnp.any(hit, axis=0, keepdims=True)

        conf_base += jnp.sum(jnp.where(ign, 0.0, conf * conf))

        eq = (c_c == a * _NPIX + p_r).astype(f32)
        raw_sel += jax.lax.dot_general(
            eq, x[base:base + _NCH, :], (((1,), (1,)), ((), ())),
            preferred_element_type=f32)
        ign_sel += jnp.sum(eq * ign.astype(f32), axis=1, keepdims=True)

    sx_sel = jax.nn.sigmoid(raw_sel[:, 0:1])
    sy_sel = jax.nn.sigmoid(raw_sel[:, 1:2])
    tw_sel = raw_sel[:, 2:3]
    th_sel = raw_sel[:, 3:4]
    conf_sel = jax.nn.sigmoid(raw_sel[:, 4:5])
    cls_sel = raw_sel[:, 5:5 + _NC]
    bx_sel = sx_sel + gi_c.astype(f32)
    by_sel = sy_sel + gj_c.astype(f32)
    bw_sel = jnp.exp(tw_sel) * awn
    bh_sel = jnp.exp(th_sel) * ahn
    cmax = jnp.max(cls_sel, axis=1, keepdims=True)
    lse_sel = jnp.log(jnp.sum(jnp.exp(cls_sel - cmax), axis=1,
                              keepdims=True)) + cmax
    clsv_sel = jnp.sum(onehot_cls * cls_sel, axis=1, keepdims=True)

    iou_sel = _iou(gx_c, gy_c, gw_c, gh_c, bx_sel, by_sel, bw_sel, bh_sel)
    conf_corr = jnp.where(
        winner,
        25.0 * (conf_sel - iou_sel) ** 2 - (1.0 - ign_sel) * conf_sel ** 2,
        0.0)
    coord_corr = jnp.where(
        winner,
        cm * cm * ((sx_sel - tc0) ** 2 + (sy_sel - tc1) ** 2
                   + (tw_sel - tc2) ** 2 + (th_sel - tc3) ** 2)
        - 1e-4 * ((sx_sel - 0.5) ** 2 + (sy_sel - 0.5) ** 2
                  + tw_sel ** 2 + th_sel ** 2),
        0.0)
    cls_corr = jnp.where(winner, -(clsv_sel - lse_sel), 0.0)

    partial = (conf_base + 1e-4 * coord_base
               + jnp.sum(conf_corr) + jnp.sum(coord_corr) + jnp.sum(cls_corr))
    out_ref[0, 0, :] = jnp.full((128,), partial, f32)


@jax.jit
def kernel(output, target):
    nB = output.shape[0]
    x = output.reshape(nB, _NA * _NCH, _NPIX)
    tb = target.reshape(nB, _NT, 5)
    tbt = jnp.transpose(tb, (0, 2, 1))
    partials = pl.pallas_call(
        _region_loss_kernel,
        grid=(nB,),
        in_specs=[
            pl.BlockSpec((1, _NA * _NCH, _NPIX), lambda i: (i, 0, 0)),
            pl.BlockSpec((1, _NT, 5), lambda i: (i, 0, 0)),
            pl.BlockSpec((1, 5, _NT), lambda i: (i, 0, 0)),
        ],
        out_specs=pl.BlockSpec((1, 1, 128), lambda i: (i, 0, 0)),
        out_shape=jax.ShapeDtypeStruct((nB, 1, 128), jnp.float32),
        compiler_params=pltpu.CompilerParams(
            dimension_semantics=("arbitrary",)),
    )(x, tb, tbt)
    return jnp.sum(partials[:, 0, 0]) / nB


# speedup vs baseline: 1.3413x; 1.3413x over previous
import functools

import jax
import jax.numpy as jnp
import numpy as np
from jax.experimental import pallas as pl
from jax.experimental.pallas import tpu as pltpu

_NC = 20
_NA = 5
_ANCHORS = np.array(
    [1.3221, 1.73145, 3.19275, 4.00944, 5.05587, 8.09892, 9.47112, 4.84053,
     11.2364, 10.0071],
    dtype=np.float32).reshape(_NA, 2)
_THRESH = 0.6
_AREA_FRAC = _THRESH / (1.0 + _THRESH)
_NH = 19
_NW = 19
_NPIX = _NH * _NW
_NT = 50
_NCH = 5 + _NC


def _iou(b1x, b1y, b1w, b1h, b2x, b2y, b2w, b2h):
    mx = jnp.minimum(b1x - b1w / 2.0, b2x - b2w / 2.0)
    Mx = jnp.maximum(b1x + b1w / 2.0, b2x + b2w / 2.0)
    my = jnp.minimum(b1y - b1h / 2.0, b2y - b2h / 2.0)
    My = jnp.maximum(b1y + b1h / 2.0, b2y + b2h / 2.0)
    uw = Mx - mx
    uh = My - my
    cw = b1w + b2w - uw
    ch = b1h + b2h - uh
    carea = jnp.where((cw <= 0) | (ch <= 0), 0.0, cw * ch)
    uarea = b1w * b1h + b2w * b2h - carea
    return carea / uarea


def _const_vec(vals, shape, dim):
    idx = jax.lax.broadcasted_iota(jnp.int32, shape, dim)
    out = jnp.zeros(shape, jnp.float32)
    for i, v in enumerate(vals):
        out = jnp.where(idx == i, float(v), out)
    return out


def _one_image(x, tb, tbt):
    f32 = jnp.float32

    xs_c = tb[:, 1:2]
    ys_c = tb[:, 2:3]
    ws_c = tb[:, 3:4]
    hs_c = tb[:, 4:5]
    cls_c = tb[:, 0:1]
    gx_c = xs_c * _NW
    gy_c = ys_c * _NH
    gw_c = ws_c * _NW
    gh_c = hs_c * _NH
    gi_c = gx_c.astype(jnp.int32)
    gj_c = gy_c.astype(jnp.int32)

    xs_r = tbt[1:2, :]
    gx_r = xs_r * _NW
    gy_r = tbt[2:3, :] * _NH
    gw_r = tbt[3:4, :] * _NW
    gh_r = tbt[4:5, :] * _NH
    gi_r = gx_r.astype(jnp.int32)
    gj_r = gy_r.astype(jnp.int32)

    ti = jax.lax.broadcasted_iota(jnp.int32, (_NT, _NT), 0)
    tj = jax.lax.broadcasted_iota(jnp.int32, (_NT, _NT), 1)
    zeros_r = (xs_r == 0.0).astype(f32)
    zeros_c = (xs_c == 0.0).astype(f32)
    cnt_c = jnp.sum(jnp.where(tj <= ti, zeros_r, 0.0), axis=1, keepdims=True)
    valid_c = cnt_c == 0.0
    cnt_r = jnp.sum(jnp.where(ti <= tj, zeros_c, 0.0), axis=0, keepdims=True)
    valid_r = cnt_r == 0.0

    aw_r = _const_vec(_ANCHORS[:, 0], (1, _NA), 1)
    ah_r = _const_vec(_ANCHORS[:, 1], (1, _NA), 1)
    aw_c = _const_vec(_ANCHORS[:, 0], (_NA, 1), 0)
    ah_c = _const_vec(_ANCHORS[:, 1], (_NA, 1), 0)
    z = jnp.zeros((), f32)
    an_idx_r = jax.lax.broadcasted_iota(jnp.int32, (_NT, _NA), 1)
    tmp_c = _iou(z, z, aw_r, ah_r, z, z, gw_c, gh_c)
    m_c = jnp.max(tmp_c, axis=1, keepdims=True)
    bn_c = jnp.min(jnp.where(tmp_c == m_c, an_idx_r, _NA), axis=1,
                   keepdims=True)
    an_idx_c = jax.lax.broadcasted_iota(jnp.int32, (_NA, _NT), 0)
    tmp_r = _iou(z, z, aw_c, ah_c, z, z, gw_r, gh_r)
    m_r = jnp.max(tmp_r, axis=0, keepdims=True)
    bn_r = jnp.min(jnp.where(tmp_r == m_r, an_idx_c, _NA), axis=0,
                   keepdims=True)

    c_c = bn_c * _NPIX + gj_c * _NW + gi_c
    c_r = bn_r * _NPIX + gj_r * _NW + gi_r
    conflict = jnp.sum(
        jnp.where((tj > ti) & valid_r & (c_r == c_c), 1.0, 0.0),
        axis=1, keepdims=True)
    winner = valid_c & (conflict == 0.0)

    onehot_bn = (an_idx_r == bn_c).astype(f32)
    awn = jnp.sum(onehot_bn * aw_r, axis=1, keepdims=True)
    ahn = jnp.sum(onehot_bn * ah_r, axis=1, keepdims=True)

    tc0 = gx_c - gi_c.astype(f32)
    tc1 = gy_c - gj_c.astype(f32)
    tc2 = jnp.log(gw_c / awn)
    tc3 = jnp.log(gh_c / ahn)
    cm = 2.0 - ws_c * hs_c
    cls_idx = cls_c.astype(jnp.int32)
    cidx_r = jax.lax.broadcasted_iota(jnp.int32, (_NT, _NC), 1)
    onehot_cls = (cidx_r == cls_idx).astype(f32)

    gx1 = jnp.where(valid_c, gx_c - gw_c * 0.5, 3.0e38)
    gx2 = gx_c + gw_c * 0.5
    gy1 = gy_c - gh_c * 0.5
    gy2 = gy_c + gh_c * 0.5
    rhs_g = _AREA_FRAC * (gw_c * gh_c)

    col = jax.lax.broadcasted_iota(jnp.int32, (1, _NPIX), 1)
    grid_x = (col % _NW).astype(f32)
    grid_y = (col // _NW).astype(f32)
    p_r = jax.lax.broadcasted_iota(jnp.int32, (_NT, _NPIX), 1)

    base_vec = jnp.zeros((1, _NPIX), f32)
    raw_sel = jnp.zeros((_NT, _NCH), f32)
    ign_sel = jnp.zeros((_NT, 1), f32)

    for a in range(_NA):
        base = a * _NCH
        tx = x[base + 0:base + 1, :]
        ty = x[base + 1:base + 2, :]
        tw = x[base + 2:base + 3, :]
        th = x[base + 3:base + 4, :]
        cf = x[base + 4:base + 5, :]

        sx = jax.nn.sigmoid(tx)
        sy = jax.nn.sigmoid(ty)
        conf = jax.nn.sigmoid(cf)
        bx = sx + grid_x
        by = sy + grid_y
        hbw = jnp.exp(tw) * (0.5 * float(_ANCHORS[a, 0]))
        hbh = jnp.exp(th) * (0.5 * float(_ANCHORS[a, 1]))
        px1 = bx - hbw
        px2 = bx + hbw
        py1 = by - hbh
        py2 = by + hbh
        rhs_p = (_AREA_FRAC * 4.0) * (hbw * hbh)

        cw = jnp.minimum(px2, gx2) - jnp.maximum(px1, gx1)
        ch = jnp.minimum(py2, gy2) - jnp.maximum(py1, gy1)
        carea = jnp.maximum(cw, 0.0) * jnp.maximum(ch, 0.0)
        hit = carea > (rhs_p + rhs_g)
        ignf = jnp.max(hit.astype(f32), axis=0, keepdims=True)

        base_vec += (jnp.where(ignf > 0.0, 0.0, conf * conf)
                     + 1e-4 * ((sx - 0.5) ** 2 + (sy - 0.5) ** 2
                               + tw * tw + th * th))

        eq = (c_c == a * _NPIX + p_r).astype(f32)
        raw_sel += jax.lax.dot_general(
            eq, x[base:base + _NCH, :], (((1,), (1,)), ((), ())),
            preferred_element_type=f32)
        ign_sel += jax.lax.dot_general(
            eq, ignf, (((1,), (1,)), ((), ())),
            preferred_element_type=f32)

    sx_sel = jax.nn.sigmoid(raw_sel[:, 0:1])
    sy_sel = jax.nn.sigmoid(raw_sel[:, 1:2])
    tw_sel = raw_sel[:, 2:3]
    th_sel = raw_sel[:, 3:4]
    conf_sel = jax.nn.sigmoid(raw_sel[:, 4:5])
    cls_sel = raw_sel[:, 5:5 + _NC]
    bx_sel = sx_sel + gi_c.astype(f32)
    by_sel = sy_sel + gj_c.astype(f32)
    bw_sel = jnp.exp(tw_sel) * awn
    bh_sel = jnp.exp(th_sel) * ahn
    cmax = jnp.max(cls_sel, axis=1, keepdims=True)
    lse_sel = jnp.log(jnp.sum(jnp.exp(cls_sel - cmax), axis=1,
                              keepdims=True)) + cmax
    clsv_sel = jnp.sum(onehot_cls * cls_sel, axis=1, keepdims=True)

    iou_sel = _iou(gx_c, gy_c, gw_c, gh_c, bx_sel, by_sel, bw_sel, bh_sel)
    conf_corr = jnp.where(
        winner,
        25.0 * (conf_sel - iou_sel) ** 2 - (1.0 - ign_sel) * conf_sel ** 2,
        0.0)
    coord_corr = jnp.where(
        winner,
        cm * cm * ((sx_sel - tc0) ** 2 + (sy_sel - tc1) ** 2
                   + (tw_sel - tc2) ** 2 + (th_sel - tc3) ** 2)
        - 1e-4 * ((sx_sel - 0.5) ** 2 + (sy_sel - 0.5) ** 2
                  + tw_sel ** 2 + th_sel ** 2),
        0.0)
    cls_corr = jnp.where(winner, -(clsv_sel - lse_sel), 0.0)

    corr = conf_corr + coord_corr + cls_corr
    return jnp.sum(base_vec) + jnp.sum(corr)


_IMG_PER = 2


def _region_loss_kernel(x_ref, tb_ref, tbt_ref, out_ref):
    partial = jnp.zeros((), jnp.float32)
    for k in range(_IMG_PER):
        partial += _one_image(x_ref[k], tb_ref[k], tbt_ref[k])
    out_ref[0, 0, :] = jnp.full((128,), partial, jnp.float32)


@jax.jit
def kernel(output, target):
    nB = output.shape[0]
    x = output.reshape(nB, _NA * _NCH, _NPIX)
    tb = target.reshape(nB, _NT, 5)
    tbt = jnp.transpose(tb, (0, 2, 1))
    ng = nB // _IMG_PER
    partials = pl.pallas_call(
        _region_loss_kernel,
        grid=(ng,),
        in_specs=[
            pl.BlockSpec((_IMG_PER, _NA * _NCH, _NPIX), lambda i: (i, 0, 0)),
            pl.BlockSpec((_IMG_PER, _NT, 5), lambda i: (i, 0, 0)),
            pl.BlockSpec((_IMG_PER, 5, _NT), lambda i: (i, 0, 0)),
        ],
        out_specs=pl.BlockSpec((1, 1, 128), lambda i: (i, 0, 0)),
        out_shape=jax.ShapeDtypeStruct((ng, 1, 128), jnp.float32),
        compiler_params=pltpu.CompilerParams(
            dimension_semantics=("arbitrary",)),
    )(x, tb, tbt)
    return jnp.sum(partials[:, 0, 0]) / nB


# speedup vs baseline: 1.3661x; 1.0185x over previous
import functools

import jax
import jax.numpy as jnp
import numpy as np
from jax.experimental import pallas as pl
from jax.experimental.pallas import tpu as pltpu

_NC = 20
_NA = 5
_ANCHORS = np.array(
    [1.3221, 1.73145, 3.19275, 4.00944, 5.05587, 8.09892, 9.47112, 4.84053,
     11.2364, 10.0071],
    dtype=np.float32).reshape(_NA, 2)
_THRESH = 0.6
_AREA_FRAC = _THRESH / (1.0 + _THRESH)
_NH = 19
_NW = 19
_NPIX = _NH * _NW
_NT = 50
_NCH = 5 + _NC


def _iou(b1x, b1y, b1w, b1h, b2x, b2y, b2w, b2h):
    mx = jnp.minimum(b1x - b1w / 2.0, b2x - b2w / 2.0)
    Mx = jnp.maximum(b1x + b1w / 2.0, b2x + b2w / 2.0)
    my = jnp.minimum(b1y - b1h / 2.0, b2y - b2h / 2.0)
    My = jnp.maximum(b1y + b1h / 2.0, b2y + b2h / 2.0)
    uw = Mx - mx
    uh = My - my
    cw = b1w + b2w - uw
    ch = b1h + b2h - uh
    carea = jnp.where((cw <= 0) | (ch <= 0), 0.0, cw * ch)
    uarea = b1w * b1h + b2w * b2h - carea
    return carea / uarea


def _const_vec(vals, shape, dim):
    idx = jax.lax.broadcasted_iota(jnp.int32, shape, dim)
    out = jnp.zeros(shape, jnp.float32)
    for i, v in enumerate(vals):
        out = jnp.where(idx == i, float(v), out)
    return out


def _one_image(x, tb, tbt):
    f32 = jnp.float32

    xs_c = tb[:, 1:2]
    ys_c = tb[:, 2:3]
    ws_c = tb[:, 3:4]
    hs_c = tb[:, 4:5]
    cls_c = tb[:, 0:1]
    gx_c = xs_c * _NW
    gy_c = ys_c * _NH
    gw_c = ws_c * _NW
    gh_c = hs_c * _NH
    gi_c = gx_c.astype(jnp.int32)
    gj_c = gy_c.astype(jnp.int32)

    gx_r = tbt[1:2, :] * _NW
    gy_r = tbt[2:3, :] * _NH
    gw_r = tbt[3:4, :] * _NW
    gh_r = tbt[4:5, :] * _NH
    gi_r = gx_r.astype(jnp.int32)
    gj_r = gy_r.astype(jnp.int32)

    dot = functools.partial(jax.lax.dot_general,
                            preferred_element_type=f32)

    ti = jax.lax.broadcasted_iota(jnp.int32, (_NT, _NT), 0)
    tj = jax.lax.broadcasted_iota(jnp.int32, (_NT, _NT), 1)
    xs_r = tbt[1:2, :]
    zeros_r = (xs_r == 0.0).astype(f32)
    zeros_c = (xs_c == 0.0).astype(f32)
    cnt_c = jnp.sum(jnp.where(tj <= ti, zeros_r, 0.0), axis=1, keepdims=True)
    valid_c = cnt_c == 0.0
    cnt_r = jnp.sum(jnp.where(ti <= tj, zeros_c, 0.0), axis=0, keepdims=True)
    valid_r = cnt_r == 0.0

    aw_r = _const_vec(_ANCHORS[:, 0], (1, _NA), 1)
    ah_r = _const_vec(_ANCHORS[:, 1], (1, _NA), 1)
    aw_c = _const_vec(_ANCHORS[:, 0], (_NA, 1), 0)
    ah_c = _const_vec(_ANCHORS[:, 1], (_NA, 1), 0)
    z = jnp.zeros((), f32)
    an_idx_r = jax.lax.broadcasted_iota(jnp.int32, (_NT, _NA), 1)
    tmp_c = _iou(z, z, aw_r, ah_r, z, z, gw_c, gh_c)
    m_c = jnp.max(tmp_c, axis=1, keepdims=True)
    bn_c = jnp.min(jnp.where(tmp_c == m_c, an_idx_r, _NA), axis=1,
                   keepdims=True)
    an_idx_c = jax.lax.broadcasted_iota(jnp.int32, (_NA, _NT), 0)
    tmp_r = _iou(z, z, aw_c, ah_c, z, z, gw_r, gh_r)
    m_r = jnp.max(tmp_r, axis=0, keepdims=True)
    bn_r = jnp.min(jnp.where(tmp_r == m_r, an_idx_c, _NA), axis=0,
                   keepdims=True)

    c_c = bn_c * _NPIX + gj_c * _NW + gi_c
    c_r = bn_r * _NPIX + gj_r * _NW + gi_r
    conflict = jnp.sum(
        jnp.where((tj > ti) & valid_r & (c_r == c_c), 1.0, 0.0),
        axis=1, keepdims=True)
    winner = valid_c & (conflict == 0.0)

    onehot_bn = (an_idx_r == bn_c).astype(f32)
    awn = jnp.sum(onehot_bn * aw_r, axis=1, keepdims=True)
    ahn = jnp.sum(onehot_bn * ah_r, axis=1, keepdims=True)

    tc0 = gx_c - gi_c.astype(f32)
    tc1 = gy_c - gj_c.astype(f32)
    tc2 = jnp.log(gw_c / awn)
    tc3 = jnp.log(gh_c / ahn)
    cm = 2.0 - ws_c * hs_c
    cls_idx = cls_c.astype(jnp.int32)
    cidx_r = jax.lax.broadcasted_iota(jnp.int32, (_NT, _NC), 1)
    onehot_cls = (cidx_r == cls_idx).astype(f32)

    gx1 = jnp.where(valid_c, gx_c - gw_c * 0.5, 3.0e38)
    gx2 = gx_c + gw_c * 0.5
    gy1 = gy_c - gh_c * 0.5
    gy2 = gy_c + gh_c * 0.5
    rhs_g = _AREA_FRAC * (gw_c * gh_c)

    col = jax.lax.broadcasted_iota(jnp.int32, (1, _NPIX), 1)
    grid_x = (col % _NW).astype(f32)
    grid_y = (col // _NW).astype(f32)
    p_r = jax.lax.broadcasted_iota(jnp.int32, (_NT, _NPIX), 1)
    d_cell = c_c - p_r
    ones_r = jnp.full((1, _NT), 1.0, f32)

    base_vec = jnp.zeros((1, _NPIX), f32)
    raw_sel = jnp.zeros((_NT, _NCH), f32)
    ign_sel = jnp.zeros((_NT, 1), f32)

    for a in range(_NA):
        base = a * _NCH
        tx = x[base + 0:base + 1, :]
        ty = x[base + 1:base + 2, :]
        tw = x[base + 2:base + 3, :]
        th = x[base + 3:base + 4, :]
        cf = x[base + 4:base + 5, :]

        sx = jax.nn.sigmoid(tx)
        sy = jax.nn.sigmoid(ty)
        conf = jax.nn.sigmoid(cf)
        bx = sx + grid_x
        by = sy + grid_y
        hbw = jnp.exp(tw) * (0.5 * float(_ANCHORS[a, 0]))
        hbh = jnp.exp(th) * (0.5 * float(_ANCHORS[a, 1]))
        px1 = bx - hbw
        px2 = bx + hbw
        py1 = by - hbh
        py2 = by + hbh
        rhs_p = (_AREA_FRAC * 4.0) * (hbw * hbh)

        cw = jnp.minimum(px2, gx2) - jnp.maximum(px1, gx1)
        ch = jnp.minimum(py2, gy2) - jnp.maximum(py1, gy1)
        carea = jnp.maximum(cw, 0.0) * ch
        hitf = jnp.where(carea > (rhs_p + rhs_g), 1.0, 0.0)
        ignf = jnp.max(hitf, axis=0, keepdims=True)

        base_vec += (jnp.where(ignf > 0.0, 0.0, conf * conf)
                     + 1e-4 * ((sx - 0.5) ** 2 + (sy - 0.5) ** 2
                               + tw * tw + th * th))

        eq = jnp.where(d_cell == a * _NPIX, 1.0, 0.0)
        raw_sel += dot(eq, x[base:base + _NCH, :], (((1,), (1,)), ((), ())))
        ign_sel += dot(eq, ignf, (((1,), (1,)), ((), ())))

    sx_sel = jax.nn.sigmoid(raw_sel[:, 0:1])
    sy_sel = jax.nn.sigmoid(raw_sel[:, 1:2])
    tw_sel = raw_sel[:, 2:3]
    th_sel = raw_sel[:, 3:4]
    conf_sel = jax.nn.sigmoid(raw_sel[:, 4:5])
    cls_sel = raw_sel[:, 5:5 + _NC]
    bx_sel = sx_sel + gi_c.astype(f32)
    by_sel = sy_sel + gj_c.astype(f32)
    bw_sel = jnp.exp(tw_sel) * awn
    bh_sel = jnp.exp(th_sel) * ahn
    cmax = jnp.max(cls_sel, axis=1, keepdims=True)
    lse_sel = jnp.log(jnp.sum(jnp.exp(cls_sel - cmax), axis=1,
                              keepdims=True)) + cmax
    clsv_sel = jnp.sum(onehot_cls * cls_sel, axis=1, keepdims=True)

    iou_sel = _iou(gx_c, gy_c, gw_c, gh_c, bx_sel, by_sel, bw_sel, bh_sel)
    conf_corr = jnp.where(
        winner,
        25.0 * (conf_sel - iou_sel) ** 2 - (1.0 - ign_sel) * conf_sel ** 2,
        0.0)
    coord_corr = jnp.where(
        winner,
        cm * cm * ((sx_sel - tc0) ** 2 + (sy_sel - tc1) ** 2
                   + (tw_sel - tc2) ** 2 + (th_sel - tc3) ** 2)
        - 1e-4 * ((sx_sel - 0.5) ** 2 + (sy_sel - 0.5) ** 2
                  + tw_sel ** 2 + th_sel ** 2),
        0.0)
    cls_corr = jnp.where(winner, -(clsv_sel - lse_sel), 0.0)

    corr = conf_corr + coord_corr + cls_corr
    return jnp.sum(base_vec) + jnp.sum(corr)


_IMG_PER = 2


def _region_loss_kernel(x_ref, tb_ref, tbt_ref, out_ref):
    partial = jnp.zeros((), jnp.float32)
    for k in range(_IMG_PER):
        partial += _one_image(x_ref[k], tb_ref[k], tbt_ref[k])
    out_ref[0, 0, :] = jnp.full((128,), partial, jnp.float32)


@jax.jit
def kernel(output, target):
    nB = output.shape[0]
    x = output.reshape(nB, _NA * _NCH, _NPIX)
    tb = target.reshape(nB, _NT, 5)
    tbt = jnp.transpose(tb, (0, 2, 1))
    ng = nB // _IMG_PER
    partials = pl.pallas_call(
        _region_loss_kernel,
        grid=(ng,),
        in_specs=[
            pl.BlockSpec((_IMG_PER, _NA * _NCH, _NPIX), lambda i: (i, 0, 0)),
            pl.BlockSpec((_IMG_PER, _NT, 5), lambda i: (i, 0, 0)),
            pl.BlockSpec((_IMG_PER, 5, _NT), lambda i: (i, 0, 0)),
        ],
        out_specs=pl.BlockSpec((1, 1, 128), lambda i: (i, 0, 0)),
        out_shape=jax.ShapeDtypeStruct((ng, 1, 128), jnp.float32),
        compiler_params=pltpu.CompilerParams(
            dimension_semantics=("parallel",)),
    )(x, tb, tbt)
    return jnp.sum(partials[:, 0, 0]) / nB


# speedup vs baseline: 1.3678x; 1.0013x over previous
import functools

import jax
import jax.numpy as jnp
import numpy as np
from jax.experimental import pallas as pl
from jax.experimental.pallas import tpu as pltpu

_NC = 20
_NA = 5
_ANCHORS = np.array(
    [1.3221, 1.73145, 3.19275, 4.00944, 5.05587, 8.09892, 9.47112, 4.84053,
     11.2364, 10.0071],
    dtype=np.float32).reshape(_NA, 2)
_THRESH = 0.6
_AREA_FRAC = _THRESH / (1.0 + _THRESH)
_NH = 19
_NW = 19
_NPIX = _NH * _NW
_NT = 50
_NCH = 5 + _NC


def _iou(b1x, b1y, b1w, b1h, b2x, b2y, b2w, b2h):
    mx = jnp.minimum(b1x - b1w / 2.0, b2x - b2w / 2.0)
    Mx = jnp.maximum(b1x + b1w / 2.0, b2x + b2w / 2.0)
    my = jnp.minimum(b1y - b1h / 2.0, b2y - b2h / 2.0)
    My = jnp.maximum(b1y + b1h / 2.0, b2y + b2h / 2.0)
    uw = Mx - mx
    uh = My - my
    cw = b1w + b2w - uw
    ch = b1h + b2h - uh
    carea = jnp.where((cw <= 0) | (ch <= 0), 0.0, cw * ch)
    uarea = b1w * b1h + b2w * b2h - carea
    return carea / uarea


def _const_vec(vals, shape, dim):
    idx = jax.lax.broadcasted_iota(jnp.int32, shape, dim)
    out = jnp.zeros(shape, jnp.float32)
    for i, v in enumerate(vals):
        out = jnp.where(idx == i, float(v), out)
    return out


def _one_image(x, tb, tbt):
    f32 = jnp.float32

    xs_c = tb[:, 1:2]
    ys_c = tb[:, 2:3]
    ws_c = tb[:, 3:4]
    hs_c = tb[:, 4:5]
    cls_c = tb[:, 0:1]
    gx_c = xs_c * _NW
    gy_c = ys_c * _NH
    gw_c = ws_c * _NW
    gh_c = hs_c * _NH
    gi_c = gx_c.astype(jnp.int32)
    gj_c = gy_c.astype(jnp.int32)

    gx_r = tbt[1:2, :] * _NW
    gy_r = tbt[2:3, :] * _NH
    gw_r = tbt[3:4, :] * _NW
    gh_r = tbt[4:5, :] * _NH
    gi_r = gx_r.astype(jnp.int32)
    gj_r = gy_r.astype(jnp.int32)

    dot = functools.partial(jax.lax.dot_general,
                            preferred_element_type=f32)

    ti = jax.lax.broadcasted_iota(jnp.int32, (_NT, _NT), 0)
    tj = jax.lax.broadcasted_iota(jnp.int32, (_NT, _NT), 1)
    zeros_r = (tbt[1:2, :] == 0.0).astype(f32)
    zeros_c = (xs_c == 0.0).astype(f32)
    cnt_c = jnp.sum(jnp.where(tj <= ti, zeros_r, 0.0), axis=1, keepdims=True)
    valid_c = cnt_c == 0.0
    cnt_r = jnp.sum(jnp.where(ti <= tj, zeros_c, 0.0), axis=0, keepdims=True)
    valid_r = cnt_r == 0.0

    aw_r = _const_vec(_ANCHORS[:, 0], (1, _NA), 1)
    ah_r = _const_vec(_ANCHORS[:, 1], (1, _NA), 1)
    aw_c = _const_vec(_ANCHORS[:, 0], (_NA, 1), 0)
    ah_c = _const_vec(_ANCHORS[:, 1], (_NA, 1), 0)
    z = jnp.zeros((), f32)
    an_idx_r = jax.lax.broadcasted_iota(jnp.int32, (_NT, _NA), 1)
    tmp_c = _iou(z, z, aw_r, ah_r, z, z, gw_c, gh_c)
    m_c = jnp.max(tmp_c, axis=1, keepdims=True)
    bn_c = jnp.min(jnp.where(tmp_c == m_c, an_idx_r, _NA), axis=1,
                   keepdims=True)
    an_idx_c = jax.lax.broadcasted_iota(jnp.int32, (_NA, _NT), 0)
    gw_r = tbt[3:4, :] * _NW
    gh_r = tbt[4:5, :] * _NH
    tmp_r = _iou(z, z, aw_c, ah_c, z, z, gw_r, gh_r)
    m_r = jnp.max(tmp_r, axis=0, keepdims=True)
    bn_r = jnp.min(jnp.where(tmp_r == m_r, an_idx_c, _NA), axis=0,
                   keepdims=True)

    gx_r = tbt[1:2, :] * _NW
    gy_r = tbt[2:3, :] * _NH
    gi_r = gx_r.astype(jnp.int32)
    gj_r = gy_r.astype(jnp.int32)
    c_c = bn_c * _NPIX + gj_c * _NW + gi_c
    c_r = bn_r * _NPIX + gj_r * _NW + gi_r
    conflict = jnp.sum(
        jnp.where((tj > ti) & valid_r & (c_r == c_c), 1.0, 0.0),
        axis=1, keepdims=True)
    winner = valid_c & (conflict == 0.0)

    onehot_bn = (an_idx_r == bn_c).astype(f32)
    awn = jnp.sum(onehot_bn * aw_r, axis=1, keepdims=True)
    ahn = jnp.sum(onehot_bn * ah_r, axis=1, keepdims=True)

    tc0 = gx_c - gi_c.astype(f32)
    tc1 = gy_c - gj_c.astype(f32)
    tc2 = jnp.log(gw_c / awn)
    tc3 = jnp.log(gh_c / ahn)
    cm = 2.0 - ws_c * hs_c
    cls_idx = cls_c.astype(jnp.int32)
    cidx_r = jax.lax.broadcasted_iota(jnp.int32, (_NT, _NC), 1)
    onehot_cls = (cidx_r == cls_idx).astype(f32)

    gx1 = jnp.where(valid_c, gx_c - gw_c * 0.5, 3.0e38)
    gx2 = gx_c + gw_c * 0.5
    gy1 = gy_c - gh_c * 0.5
    gy2 = gy_c + gh_c * 0.5
    rhs_g = _AREA_FRAC * (gw_c * gh_c)

    col = jax.lax.broadcasted_iota(jnp.int32, (1, _NPIX), 1)
    grid_x = (col % _NW).astype(f32)
    grid_y = (col // _NW).astype(f32)
    p_r = jax.lax.broadcasted_iota(jnp.int32, (_NT, _NPIX), 1)
    d_cell = c_c - p_r

    base_vec = jnp.zeros((1, _NPIX), f32)
    raw_sel = jnp.zeros((_NT, _NCH), f32)
    ign_sel = jnp.zeros((_NT, 1), f32)

    for a in range(_NA):
        base = a * _NCH
        tx = x[base + 0:base + 1, :]
        ty = x[base + 1:base + 2, :]
        tw = x[base + 2:base + 3, :]
        th = x[base + 3:base + 4, :]
        cf = x[base + 4:base + 5, :]

        sx = jax.nn.sigmoid(tx)
        sy = jax.nn.sigmoid(ty)
        conf = jax.nn.sigmoid(cf)
        bx = sx + grid_x
        by = sy + grid_y
        hbw = jnp.exp(tw) * (0.5 * float(_ANCHORS[a, 0]))
        hbh = jnp.exp(th) * (0.5 * float(_ANCHORS[a, 1]))
        px1 = bx - hbw
        px2 = bx + hbw
        py1 = by - hbh
        py2 = by + hbh
        rhs_p = (_AREA_FRAC * 4.0) * (hbw * hbh)

        cw = jnp.minimum(px2, gx2) - jnp.maximum(px1, gx1)
        ch = jnp.minimum(py2, gy2) - jnp.maximum(py1, gy1)
        carea = jnp.maximum(cw, 0.0) * ch
        hitf = jnp.where(carea > (rhs_p + rhs_g), 1.0, 0.0)
        ignf = jnp.max(hitf, axis=0, keepdims=True)

        base_vec += (jnp.where(ignf > 0.0, 0.0, conf * conf)
                     + 1e-4 * ((sx - 0.5) ** 2 + (sy - 0.5) ** 2
                               + tw * tw + th * th))

        eq = jnp.where(d_cell == a * _NPIX, 1.0, 0.0)
        raw_sel += dot(eq, x[base:base + _NCH, :], (((1,), (1,)), ((), ())))
        ign_sel += dot(eq, ignf, (((1,), (1,)), ((), ())))

    sx_sel = jax.nn.sigmoid(raw_sel[:, 0:1])
    sy_sel = jax.nn.sigmoid(raw_sel[:, 1:2])
    tw_sel = raw_sel[:, 2:3]
    th_sel = raw_sel[:, 3:4]
    conf_sel = jax.nn.sigmoid(raw_sel[:, 4:5])
    cls_sel = raw_sel[:, 5:5 + _NC]
    bx_sel = sx_sel + gi_c.astype(f32)
    by_sel = sy_sel + gj_c.astype(f32)
    bw_sel = jnp.exp(tw_sel) * awn
    bh_sel = jnp.exp(th_sel) * ahn
    cmax = jnp.max(cls_sel, axis=1, keepdims=True)
    lse_sel = jnp.log(jnp.sum(jnp.exp(cls_sel - cmax), axis=1,
                              keepdims=True)) + cmax
    clsv_sel = jnp.sum(onehot_cls * cls_sel, axis=1, keepdims=True)

    iou_sel = _iou(gx_c, gy_c, gw_c, gh_c, bx_sel, by_sel, bw_sel, bh_sel)
    conf_corr = jnp.where(
        winner,
        25.0 * (conf_sel - iou_sel) ** 2 - (1.0 - ign_sel) * conf_sel ** 2,
        0.0)
    coord_corr = jnp.where(
        winner,
        cm * cm * ((sx_sel - tc0) ** 2 + (sy_sel - tc1) ** 2
                   + (tw_sel - tc2) ** 2 + (th_sel - tc3) ** 2)
        - 1e-4 * ((sx_sel - 0.5) ** 2 + (sy_sel - 0.5) ** 2
                  + tw_sel ** 2 + th_sel ** 2),
        0.0)
    cls_corr = jnp.where(winner, -(clsv_sel - lse_sel), 0.0)

    corr = conf_corr + coord_corr + cls_corr
    return jnp.sum(base_vec) + jnp.sum(corr)


_IMG_PER = 2


def _region_loss_kernel(x_ref, tb_ref, tbt_ref, out_ref):
    partial = jnp.zeros((), jnp.float32)
    for k in range(_IMG_PER):
        partial += _one_image(x_ref[k], tb_ref[k], tbt_ref[k])
    out_ref[0, 0, :] = jnp.full((128,), partial, jnp.float32)


@jax.jit
def kernel(output, target):
    nB = output.shape[0]
    x = output.reshape(nB, _NA * _NCH, _NPIX)
    tb = target.reshape(nB, _NT, 5)
    tbt = jnp.transpose(tb, (0, 2, 1))
    ng = nB // _IMG_PER
    partials = pl.pallas_call(
        _region_loss_kernel,
        grid=(ng,),
        in_specs=[
            pl.BlockSpec((_IMG_PER, _NA * _NCH, _NPIX), lambda i: (i, 0, 0)),
            pl.BlockSpec((_IMG_PER, _NT, 5), lambda i: (i, 0, 0)),
            pl.BlockSpec((_IMG_PER, 5, _NT), lambda i: (i, 0, 0)),
        ],
        out_specs=pl.BlockSpec((1, 1, 128), lambda i: (i, 0, 0)),
        out_shape=jax.ShapeDtypeStruct((ng, 1, 128), jnp.float32),
        compiler_params=pltpu.CompilerParams(
            dimension_semantics=("parallel",)),
    )(x, tb, tbt)
    return jnp.sum(partials[:, 0, 0]) / nB


# speedup vs baseline: 1.3682x; 1.0003x over previous
import functools

import jax
import jax.numpy as jnp
import numpy as np
from jax.experimental import pallas as pl
from jax.experimental.pallas import tpu as pltpu

_NC = 20
_NA = 5
_ANCHORS = np.array(
    [1.3221, 1.73145, 3.19275, 4.00944, 5.05587, 8.09892, 9.47112, 4.84053,
     11.2364, 10.0071],
    dtype=np.float32).reshape(_NA, 2)
_THRESH = 0.6
_AREA_FRAC = _THRESH / (1.0 + _THRESH)
_NH = 19
_NW = 19
_NPIX = _NH * _NW
_NT = 50
_NCH = 5 + _NC
_IMG_PER = 2


def _iou(b1x, b1y, b1w, b1h, b2x, b2y, b2w, b2h):
    mx = jnp.minimum(b1x - b1w / 2.0, b2x - b2w / 2.0)
    Mx = jnp.maximum(b1x + b1w / 2.0, b2x + b2w / 2.0)
    my = jnp.minimum(b1y - b1h / 2.0, b2y - b2h / 2.0)
    My = jnp.maximum(b1y + b1h / 2.0, b2y + b2h / 2.0)
    uw = Mx - mx
    uh = My - my
    cw = b1w + b2w - uw
    ch = b1h + b2h - uh
    carea = jnp.where((cw <= 0) | (ch <= 0), 0.0, cw * ch)
    uarea = b1w * b1h + b2w * b2h - carea
    return carea / uarea


def _const_vec(vals, shape, dim):
    idx = jax.lax.broadcasted_iota(jnp.int32, shape, dim)
    out = jnp.zeros(shape, jnp.float32)
    for i, v in enumerate(vals):
        out = jnp.where(idx == i, float(v), out)
    return out


_dot = functools.partial(jax.lax.dot_general, preferred_element_type=jnp.float32)


def _gt_prep(tb, tbt):
    f32 = jnp.float32
    pr = {}

    xs_c = tb[:, 1:2]
    ws_c = tb[:, 3:4]
    hs_c = tb[:, 4:5]
    cls_c = tb[:, 0:1]
    gx_c = xs_c * _NW
    gy_c = tb[:, 2:3] * _NH
    gw_c = ws_c * _NW
    gh_c = hs_c * _NH
    gi_c = gx_c.astype(jnp.int32)
    gj_c = gy_c.astype(jnp.int32)
    pr.update(gx_c=gx_c, gy_c=gy_c, gw_c=gw_c, gh_c=gh_c)

    ti = jax.lax.broadcasted_iota(jnp.int32, (_NT, _NT), 0)
    tj = jax.lax.broadcasted_iota(jnp.int32, (_NT, _NT), 1)
    zeros_r = (tbt[1:2, :] == 0.0).astype(f32)
    zeros_c = (xs_c == 0.0).astype(f32)
    cnt_c = jnp.sum(jnp.where(tj <= ti, zeros_r, 0.0), axis=1, keepdims=True)
    valid_c = cnt_c == 0.0
    cnt_r = jnp.sum(jnp.where(ti <= tj, zeros_c, 0.0), axis=0, keepdims=True)
    valid_r = cnt_r == 0.0

    aw_r = _const_vec(_ANCHORS[:, 0], (1, _NA), 1)
    ah_r = _const_vec(_ANCHORS[:, 1], (1, _NA), 1)
    aw_c = _const_vec(_ANCHORS[:, 0], (_NA, 1), 0)
    ah_c = _const_vec(_ANCHORS[:, 1], (_NA, 1), 0)
    z = jnp.zeros((), f32)
    an_idx_r = jax.lax.broadcasted_iota(jnp.int32, (_NT, _NA), 1)
    tmp_c = _iou(z, z, aw_r, ah_r, z, z, gw_c, gh_c)
    m_c = jnp.max(tmp_c, axis=1, keepdims=True)
    bn_c = jnp.min(jnp.where(tmp_c == m_c, an_idx_r, _NA), axis=1,
                   keepdims=True)
    an_idx_c = jax.lax.broadcasted_iota(jnp.int32, (_NA, _NT), 0)
    gw_r = tbt[3:4, :] * _NW
    gh_r = tbt[4:5, :] * _NH
    tmp_r = _iou(z, z, aw_c, ah_c, z, z, gw_r, gh_r)
    m_r = jnp.max(tmp_r, axis=0, keepdims=True)
    bn_r = jnp.min(jnp.where(tmp_r == m_r, an_idx_c, _NA), axis=0,
                   keepdims=True)

    gi_r = (tbt[1:2, :] * _NW).astype(jnp.int32)
    gj_r = (tbt[2:3, :] * _NH).astype(jnp.int32)
    c_c = bn_c * _NPIX + gj_c * _NW + gi_c
    c_r = bn_r * _NPIX + gj_r * _NW + gi_r
    conflict = jnp.sum(
        jnp.where((tj > ti) & valid_r & (c_r == c_c), 1.0, 0.0),
        axis=1, keepdims=True)
    pr['winner'] = valid_c & (conflict == 0.0)

    onehot_bn = (an_idx_r == bn_c).astype(f32)
    awn = jnp.sum(onehot_bn * aw_r, axis=1, keepdims=True)
    ahn = jnp.sum(onehot_bn * ah_r, axis=1, keepdims=True)
    pr.update(awn=awn, ahn=ahn)

    pr['tc0'] = gx_c - gi_c.astype(f32)
    pr['tc1'] = gy_c - gj_c.astype(f32)
    pr['tc2'] = jnp.log(gw_c / awn)
    pr['tc3'] = jnp.log(gh_c / ahn)
    pr['cm'] = 2.0 - ws_c * hs_c
    cidx_r = jax.lax.broadcasted_iota(jnp.int32, (_NT, _NC), 1)
    pr['onehot_cls'] = (cidx_r == cls_c.astype(jnp.int32)).astype(f32)
    pr['gi_f'] = gi_c.astype(f32)
    pr['gj_f'] = gj_c.astype(f32)

    pr['gx1'] = jnp.where(valid_c, gx_c - gw_c * 0.5, 3.0e38)
    pr['gx2'] = gx_c + gw_c * 0.5
    pr['gy1'] = gy_c - gh_c * 0.5
    pr['gy2'] = gy_c + gh_c * 0.5
    pr['rhs_g'] = _AREA_FRAC * (gw_c * gh_c)

    p_r = jax.lax.broadcasted_iota(jnp.int32, (_NT, _NPIX), 1)
    pr['d_cell'] = c_c - p_r
    return pr


_FENCE_KEYS = ('gx1', 'gx2', 'gy1', 'gy2', 'rhs_g', 'd_cell')


def _anchor_pass(x, a, pr, accs):
    f32 = jnp.float32
    base_vec, raw_sel, ign_sel = accs
    base = a * _NCH

    col = jax.lax.broadcasted_iota(jnp.int32, (1, _NPIX), 1)
    grid_x = (col % _NW).astype(f32)
    grid_y = (col // _NW).astype(f32)

    tx = x[base + 0:base + 1, :]
    ty = x[base + 1:base + 2, :]
    tw = x[base + 2:base + 3, :]
    th = x[base + 3:base + 4, :]
    cf = x[base + 4:base + 5, :]

    sx = jax.nn.sigmoid(tx)
    sy = jax.nn.sigmoid(ty)
    conf = jax.nn.sigmoid(cf)
    bx = sx + grid_x
    by = sy + grid_y
    hbw = jnp.exp(tw) * (0.5 * float(_ANCHORS[a, 0]))
    hbh = jnp.exp(th) * (0.5 * float(_ANCHORS[a, 1]))
    px1 = bx - hbw
    px2 = bx + hbw
    py1 = by - hbh
    py2 = by + hbh
    rhs_p = (_AREA_FRAC * 4.0) * (hbw * hbh)

    cw = jnp.minimum(px2, pr['gx2']) - jnp.maximum(px1, pr['gx1'])
    ch = jnp.minimum(py2, pr['gy2']) - jnp.maximum(py1, pr['gy1'])
    carea = jnp.maximum(cw, 0.0) * ch
    hitf = jnp.where(carea > (rhs_p + pr['rhs_g']), 1.0, 0.0)
    ignf = jnp.max(hitf, axis=0, keepdims=True)

    base_vec = base_vec + (jnp.where(ignf > 0.0, 0.0, conf * conf)
                           + 1e-4 * ((sx - 0.5) ** 2 + (sy - 0.5) ** 2
                                     + tw * tw + th * th))

    eq = jnp.where(pr['d_cell'] == a * _NPIX, 1.0, 0.0)
    raw_sel = raw_sel + _dot(eq, x[base:base + _NCH, :],
                             (((1,), (1,)), ((), ())))
    ign_sel = ign_sel + _dot(eq, ignf, (((1,), (1,)), ((), ())))
    return base_vec, raw_sel, ign_sel


def _tail(pr, accs):
    f32 = jnp.float32
    base_vec, raw_sel, ign_sel = accs
    winner = pr['winner']

    sx_sel = jax.nn.sigmoid(raw_sel[:, 0:1])
    sy_sel = jax.nn.sigmoid(raw_sel[:, 1:2])
    tw_sel = raw_sel[:, 2:3]
    th_sel = raw_sel[:, 3:4]
    conf_sel = jax.nn.sigmoid(raw_sel[:, 4:5])
    cls_sel = raw_sel[:, 5:5 + _NC]
    bx_sel = sx_sel + pr['gi_f']
    by_sel = sy_sel + pr['gj_f']
    bw_sel = jnp.exp(tw_sel) * pr['awn']
    bh_sel = jnp.exp(th_sel) * pr['ahn']
    cmax = jnp.max(cls_sel, axis=1, keepdims=True)
    lse_sel = jnp.log(jnp.sum(jnp.exp(cls_sel - cmax), axis=1,
                              keepdims=True)) + cmax
    clsv_sel = jnp.sum(pr['onehot_cls'] * cls_sel, axis=1, keepdims=True)

    iou_sel = _iou(pr['gx_c'], pr['gy_c'], pr['gw_c'], pr['gh_c'],
                   bx_sel, by_sel, bw_sel, bh_sel)
    conf_corr = jnp.where(
        winner,
        25.0 * (conf_sel - iou_sel) ** 2 - (1.0 - ign_sel) * conf_sel ** 2,
        0.0)
    coord_corr = jnp.where(
        winner,
        pr['cm'] * pr['cm'] * (
            (sx_sel - pr['tc0']) ** 2 + (sy_sel - pr['tc1']) ** 2
            + (tw_sel - pr['tc2']) ** 2 + (th_sel - pr['tc3']) ** 2)
        - 1e-4 * ((sx_sel - 0.5) ** 2 + (sy_sel - 0.5) ** 2
                  + tw_sel ** 2 + th_sel ** 2),
        0.0)
    cls_corr = jnp.where(winner, -(clsv_sel - lse_sel), 0.0)

    corr = conf_corr + coord_corr + cls_corr
    return jnp.sum(base_vec) + jnp.sum(corr)


def _one_image(x, tb, tbt):
    f32 = jnp.float32
    pr = _gt_prep(tb, tbt)
    accs = (jnp.zeros((1, _NPIX), f32), jnp.zeros((_NT, _NCH), f32),
            jnp.zeros((_NT, 1), f32))
    for a in range(_NA):
        accs = _anchor_pass(x, a, pr, accs)
    return _tail(pr, accs)


def _region_loss_kernel(x_ref, tb_ref, tbt_ref, out_ref):
    partial = jnp.zeros((), jnp.float32)
    for k in range(_IMG_PER):
        partial += _one_image(x_ref[k], tb_ref[k], tbt_ref[k])
    out_ref[0, 0, :] = jnp.full((128,), partial, jnp.float32)


@jax.jit
def kernel(output, target):
    nB = output.shape[0]
    x = output.reshape(nB, _NA * _NCH, _NPIX)
    tb = target.reshape(nB, _NT, 5)
    tbt = jnp.transpose(tb, (0, 2, 1))
    ng = nB // _IMG_PER
    partials = pl.pallas_call(
        _region_loss_kernel,
        grid=(ng,),
        in_specs=[
            pl.BlockSpec((_IMG_PER, _NA * _NCH, _NPIX), lambda i: (i, 0, 0)),
            pl.BlockSpec((_IMG_PER, _NT, 5), lambda i: (i, 0, 0)),
            pl.BlockSpec((_IMG_PER, 5, _NT), lambda i: (i, 0, 0)),
        ],
        out_specs=pl.BlockSpec((1, 1, 128), lambda i: (i, 0, 0)),
        out_shape=jax.ShapeDtypeStruct((ng, 1, 128), jnp.float32),
        compiler_params=pltpu.CompilerParams(
            dimension_semantics=("parallel",)),
    )(x, tb, tbt)
    return jnp.sum(partials[:, 0, 0]) / nB


# speedup vs baseline: 1.3684x; 1.0002x over previous
import functools

import jax
import jax.numpy as jnp
import numpy as np
from jax.experimental import pallas as pl
from jax.experimental.pallas import tpu as pltpu

_NC = 20
_NA = 5
_ANCHORS = np.array(
    [1.3221, 1.73145, 3.19275, 4.00944, 5.05587, 8.09892, 9.47112, 4.84053,
     11.2364, 10.0071],
    dtype=np.float32).reshape(_NA, 2)
_THRESH = 0.6
_AREA_FRAC = _THRESH / (1.0 + _THRESH)
_NH = 19
_NW = 19
_NPIX = _NH * _NW
_NT = 50
_NCH = 5 + _NC
_IMG_PER = 2


def _iou(b1x, b1y, b1w, b1h, b2x, b2y, b2w, b2h):
    mx = jnp.minimum(b1x - b1w / 2.0, b2x - b2w / 2.0)
    Mx = jnp.maximum(b1x + b1w / 2.0, b2x + b2w / 2.0)
    my = jnp.minimum(b1y - b1h / 2.0, b2y - b2h / 2.0)
    My = jnp.maximum(b1y + b1h / 2.0, b2y + b2h / 2.0)
    uw = Mx - mx
    uh = My - my
    cw = b1w + b2w - uw
    ch = b1h + b2h - uh
    carea = jnp.where((cw <= 0) | (ch <= 0), 0.0, cw * ch)
    uarea = b1w * b1h + b2w * b2h - carea
    return carea / uarea


def _const_vec(vals, shape, dim):
    idx = jax.lax.broadcasted_iota(jnp.int32, shape, dim)
    out = jnp.zeros(shape, jnp.float32)
    for i, v in enumerate(vals):
        out = jnp.where(idx == i, float(v), out)
    return out


_dot = functools.partial(jax.lax.dot_general, preferred_element_type=jnp.float32)


def _gt_prep(tb, tbt):
    f32 = jnp.float32
    pr = {}

    xs_c = tb[:, 1:2]
    ws_c = tb[:, 3:4]
    hs_c = tb[:, 4:5]
    cls_c = tb[:, 0:1]
    gx_c = xs_c * _NW
    gy_c = tb[:, 2:3] * _NH
    gw_c = ws_c * _NW
    gh_c = hs_c * _NH
    gi_c = gx_c.astype(jnp.int32)
    gj_c = gy_c.astype(jnp.int32)
    pr.update(gx_c=gx_c, gy_c=gy_c, gw_c=gw_c, gh_c=gh_c)

    ti = jax.lax.broadcasted_iota(jnp.int32, (_NT, _NT), 0)
    tj = jax.lax.broadcasted_iota(jnp.int32, (_NT, _NT), 1)
    zeros_r = (tbt[1:2, :] == 0.0).astype(f32)
    zeros_c = (xs_c == 0.0).astype(f32)
    cnt_c = jnp.sum(jnp.where(tj <= ti, zeros_r, 0.0), axis=1, keepdims=True)
    valid_c = cnt_c == 0.0
    cnt_r = jnp.sum(jnp.where(ti <= tj, zeros_c, 0.0), axis=0, keepdims=True)
    valid_r = cnt_r == 0.0

    aw_r = _const_vec(_ANCHORS[:, 0], (1, _NA), 1)
    ah_r = _const_vec(_ANCHORS[:, 1], (1, _NA), 1)
    aw_c = _const_vec(_ANCHORS[:, 0], (_NA, 1), 0)
    ah_c = _const_vec(_ANCHORS[:, 1], (_NA, 1), 0)
    z = jnp.zeros((), f32)
    an_idx_r = jax.lax.broadcasted_iota(jnp.int32, (_NT, _NA), 1)
    tmp_c = _iou(z, z, aw_r, ah_r, z, z, gw_c, gh_c)
    m_c = jnp.max(tmp_c, axis=1, keepdims=True)
    bn_c = jnp.min(jnp.where(tmp_c == m_c, an_idx_r, _NA), axis=1,
                   keepdims=True)
    an_idx_c = jax.lax.broadcasted_iota(jnp.int32, (_NA, _NT), 0)
    gw_r = tbt[3:4, :] * _NW
    gh_r = tbt[4:5, :] * _NH
    tmp_r = _iou(z, z, aw_c, ah_c, z, z, gw_r, gh_r)
    m_r = jnp.max(tmp_r, axis=0, keepdims=True)
    bn_r = jnp.min(jnp.where(tmp_r == m_r, an_idx_c, _NA), axis=0,
                   keepdims=True)

    gi_r = (tbt[1:2, :] * _NW).astype(jnp.int32)
    gj_r = (tbt[2:3, :] * _NH).astype(jnp.int32)
    c_c = bn_c * _NPIX + gj_c * _NW + gi_c
    c_r = bn_r * _NPIX + gj_r * _NW + gi_r
    conflict = jnp.sum(
        jnp.where((tj > ti) & valid_r & (c_r == c_c), 1.0, 0.0),
        axis=1, keepdims=True)
    pr['winner'] = valid_c & (conflict == 0.0)

    onehot_bn = (an_idx_r == bn_c).astype(f32)
    awn = jnp.sum(onehot_bn * aw_r, axis=1, keepdims=True)
    ahn = jnp.sum(onehot_bn * ah_r, axis=1, keepdims=True)
    pr.update(awn=awn, ahn=ahn)

    pr['tc0'] = gx_c - gi_c.astype(f32)
    pr['tc1'] = gy_c - gj_c.astype(f32)
    pr['tc2'] = jnp.log(gw_c / awn)
    pr['tc3'] = jnp.log(gh_c / ahn)
    pr['cm'] = 2.0 - ws_c * hs_c
    cidx_r = jax.lax.broadcasted_iota(jnp.int32, (_NT, _NC), 1)
    pr['onehot_cls'] = (cidx_r == cls_c.astype(jnp.int32)).astype(f32)
    pr['gi_f'] = gi_c.astype(f32)
    pr['gj_f'] = gj_c.astype(f32)

    pr['gx1'] = jnp.where(valid_c, gx_c - gw_c * 0.5, 3.0e38)
    pr['gx2'] = gx_c + gw_c * 0.5
    pr['gy1'] = gy_c - gh_c * 0.5
    pr['gy2'] = gy_c + gh_c * 0.5
    pr['rhs_g'] = _AREA_FRAC * (gw_c * gh_c)

    p_r = jax.lax.broadcasted_iota(jnp.int32, (_NT, _NPIX), 1)
    pr['d_cell'] = c_c - p_r
    return pr


def _anchor_pass(x, a, pr, accs):
    f32 = jnp.float32
    base_vec, raw_sel, ign_sel = accs
    base = a * _NCH

    col = jax.lax.broadcasted_iota(jnp.int32, (1, _NPIX), 1)
    grid_x = (col % _NW).astype(f32)
    grid_y = (col // _NW).astype(f32)

    tx = x[base + 0:base + 1, :]
    ty = x[base + 1:base + 2, :]
    tw = x[base + 2:base + 3, :]
    th = x[base + 3:base + 4, :]
    cf = x[base + 4:base + 5, :]

    sx = jax.nn.sigmoid(tx)
    sy = jax.nn.sigmoid(ty)
    conf = jax.nn.sigmoid(cf)
    bx = sx + grid_x
    by = sy + grid_y
    hbw = jnp.exp(tw) * (0.5 * float(_ANCHORS[a, 0]))
    hbh = jnp.exp(th) * (0.5 * float(_ANCHORS[a, 1]))
    px1 = bx - hbw
    px2 = bx + hbw
    py1 = by - hbh
    py2 = by + hbh
    rhs_p = (_AREA_FRAC * 4.0) * (hbw * hbh)

    cw = jnp.minimum(px2, pr['gx2']) - jnp.maximum(px1, pr['gx1'])
    ch = jnp.minimum(py2, pr['gy2']) - jnp.maximum(py1, pr['gy1'])
    carea = jnp.maximum(cw, 0.0) * ch
    hitf = jnp.where(carea > (rhs_p + pr['rhs_g']), 1.0, 0.0)
    ignf = jnp.max(hitf, axis=0, keepdims=True)

    base_vec = base_vec + (jnp.where(ignf > 0.0, 0.0, conf * conf)
                           + 1e-4 * ((sx - 0.5) ** 2 + (sy - 0.5) ** 2
                                     + tw * tw + th * th))

    eq = jnp.where(pr['d_cell'] == a * _NPIX, 1.0, 0.0)
    raw_sel = raw_sel + _dot(eq, x[base:base + _NCH, :],
                             (((1,), (1,)), ((), ())))
    ign_sel = ign_sel + _dot(eq, ignf, (((1,), (1,)), ((), ())))
    return base_vec, raw_sel, ign_sel


def _tail(pr, accs):
    f32 = jnp.float32
    base_vec, raw_sel, ign_sel = accs
    winner = pr['winner']

    sx_sel = jax.nn.sigmoid(raw_sel[:, 0:1])
    sy_sel = jax.nn.sigmoid(raw_sel[:, 1:2])
    tw_sel = raw_sel[:, 2:3]
    th_sel = raw_sel[:, 3:4]
    conf_sel = jax.nn.sigmoid(raw_sel[:, 4:5])
    cls_sel = raw_sel[:, 5:5 + _NC]
    bx_sel = sx_sel + pr['gi_f']
    by_sel = sy_sel + pr['gj_f']
    bw_sel = jnp.exp(tw_sel) * pr['awn']
    bh_sel = jnp.exp(th_sel) * pr['ahn']
    cmax = jnp.max(cls_sel, axis=1, keepdims=True)
    lse_sel = jnp.log(jnp.sum(jnp.exp(cls_sel - cmax), axis=1,
                              keepdims=True)) + cmax
    clsv_sel = jnp.sum(pr['onehot_cls'] * cls_sel, axis=1, keepdims=True)

    iou_sel = _iou(pr['gx_c'], pr['gy_c'], pr['gw_c'], pr['gh_c'],
                   bx_sel, by_sel, bw_sel, bh_sel)
    conf_corr = jnp.where(
        winner,
        25.0 * (conf_sel - iou_sel) ** 2 - (1.0 - ign_sel) * conf_sel ** 2,
        0.0)
    coord_corr = jnp.where(
        winner,
        pr['cm'] * pr['cm'] * (
            (sx_sel - pr['tc0']) ** 2 + (sy_sel - pr['tc1']) ** 2
            + (tw_sel - pr['tc2']) ** 2 + (th_sel - pr['tc3']) ** 2)
        - 1e-4 * ((sx_sel - 0.5) ** 2 + (sy_sel - 0.5) ** 2
                  + tw_sel ** 2 + th_sel ** 2),
        0.0)
    cls_corr = jnp.where(winner, -(clsv_sel - lse_sel), 0.0)

    corr = conf_corr + coord_corr + cls_corr
    return jnp.sum(base_vec) + jnp.sum(corr)


def _one_image(x, tb, tbt):
    f32 = jnp.float32
    pr = _gt_prep(tb, tbt)
    accs = (jnp.zeros((1, _NPIX), f32), jnp.zeros((_NT, _NCH), f32),
            jnp.zeros((_NT, 1), f32))
    for a in range(_NA):
        accs = _anchor_pass(x, a, pr, accs)
    return _tail(pr, accs)


def _region_loss_kernel(x_ref, tb_ref, tbt_ref, out_ref):
    partial = jnp.zeros((), jnp.float32)
    for k in range(_IMG_PER):
        partial += _one_image(x_ref[k], tb_ref[k], tbt_ref[k])
    out_ref[0, 0, :] = jnp.full((128,), partial, jnp.float32)


@jax.jit
def kernel(output, target):
    nB = output.shape[0]
    x = output.reshape(nB, _NA * _NCH, _NPIX)
    tb = target.reshape(nB, _NT, 5)
    tbt = jnp.transpose(tb, (0, 2, 1))
    ng = nB // _IMG_PER
    partials = pl.pallas_call(
        _region_loss_kernel,
        grid=(ng,),
        in_specs=[
            pl.BlockSpec((_IMG_PER, _NA * _NCH, _NPIX), lambda i: (i, 0, 0)),
            pl.BlockSpec((_IMG_PER, _NT, 5), lambda i: (i, 0, 0)),
            pl.BlockSpec((_IMG_PER, 5, _NT), lambda i: (i, 0, 0)),
        ],
        out_specs=pl.BlockSpec((1, 1, 128), lambda i: (i, 0, 0)),
        out_shape=jax.ShapeDtypeStruct((ng, 1, 128), jnp.float32),
        compiler_params=pltpu.CompilerParams(
            dimension_semantics=("parallel",)),
    )(x, tb, tbt)
    return jnp.sum(partials[:, 0, 0]) / nB
